# Initial kernel scaffold; baseline (speedup 1.0000x reference)
import jax
import jax.numpy as jnp
from jax.experimental import pallas as pl


def kernel(x, adj_idx, adj_val, adj_knn_idx, adj_knn_val, adj_diff_idx, adj_diff_val, W1, W2, W3):
    raise NotImplementedError("write your pallas kernel here")


# speedup vs baseline: 1.7679x; 1.7679x over previous
import functools

import jax
import jax.numpy as jnp
from jax import lax
from jax.experimental import pallas as pl
from jax.experimental.pallas import tpu as pltpu
from jax.experimental.pallas import tpu_sc as plsc

N = 10000
NP = 10240
E = 160000
NB = 80
EB = 128
E_PAD = 16 * NB * EB
BM = 2048
N_TILE = 16
ROWS = NP // N_TILE


def _mm_kernel(a_ref, w_ref, o_ref):
    k = pl.program_id(2)

    @pl.when(k == 0)
    def _():
        o_ref[...] = jnp.zeros_like(o_ref)

    o_ref[...] += jnp.dot(a_ref[...], w_ref[...],
                          preferred_element_type=jnp.float32)


def _mm_x(x, w, c_out):
    k_dim = x.shape[1]
    gm = NP // BM
    return pl.pallas_call(
        _mm_kernel,
        grid=(gm, c_out, 1),
        in_specs=[
            pl.BlockSpec((BM, k_dim), lambda i, j, k: (i, 0)),
            pl.BlockSpec((k_dim, 128), lambda i, j, k: (0, j)),
        ],
        out_specs=pl.BlockSpec((BM, 128), lambda i, j, k: (j * (NP // BM) + i, 0)),
        out_shape=jax.ShapeDtypeStruct((4 * NP, 128), jnp.float32),
    )(x, w)


def _mm_flat(h, w, c_in, c_out):
    gm = NP // BM
    return pl.pallas_call(
        _mm_kernel,
        grid=(gm, c_out, c_in),
        in_specs=[
            pl.BlockSpec((BM, 128), lambda i, j, k: (k * (NP // BM) + i, 0)),
            pl.BlockSpec((128, 128), lambda i, j, k: (k, j)),
        ],
        out_specs=pl.BlockSpec((BM, 128), lambda i, j, k: (j * (NP // BM) + i, 0)),
        out_shape=jax.ShapeDtypeStruct((4 * NP, 128), jnp.float32),
    )(h, w)


def _make_spmm():
    mesh = plsc.VectorSubcoreMesh(core_axis_name="c", subcore_axis_name="s")

    @functools.partial(
        pl.kernel,
        mesh=mesh,
        out_type=jax.ShapeDtypeStruct((4 * NP, 128), jnp.float32),
        scratch_types=[
            pltpu.VMEM((EB,), jnp.int32),
            pltpu.VMEM((EB,), jnp.int32),
            pltpu.VMEM((EB,), jnp.float32),
            pltpu.VMEM((16,), jnp.int32),
            pltpu.VMEM((EB, 128), jnp.float32),
            pltpu.VMEM_SHARED((NP, 128), jnp.float32),
            pltpu.SemaphoreType.DMA,
        ],
    )
    def spmm(sup, src3, dst3, val3, zeros_hbm, cc_hbm, out,
             idx_v, dst_v, val_v, cc_v, rows_v, accum, sem):
        cid = lax.axis_index("c")
        sid = lax.axis_index("s")
        row0 = sid * ROWS

        pltpu.sync_copy(cc_hbm, cc_v)
        c_chunks = cc_v[pl.ds(0, 16)][0]
        rounds = (c_chunks + 1) // 2

        def round_body(r, carry):
            j = r * 2 + cid
            active = j < c_chunks

            pltpu.sync_copy(zeros_hbm.at[pl.ds(row0, ROWS)],
                            accum.at[pl.ds(row0, ROWS)])
            plsc.subcore_barrier()

            @pl.when(active)
            def _():
                def edge_block(b, carry2):
                    pltpu.sync_copy(src3.at[sid, b], idx_v)
                    pltpu.sync_copy(dst3.at[sid, b], dst_v)
                    pltpu.sync_copy(val3.at[sid, b], val_v)
                    off = j * NP
                    for q in range(EB // 16):
                        sl = pl.ds(q * 16, 16)
                        idx_v[sl] = idx_v[sl] + off
                    pltpu.async_copy(sup.at[idx_v], rows_v, sem).wait()

                    def scale_16(e16, c2):
                        vvec = val_v[pl.ds(e16 * 16, 16)]
                        for i in range(16):
                            s = vvec[i]
                            e = e16 * 16 + i
                            for q in range(128 // 16):
                                sl = pl.ds(q * 16, 16)
                                rows_v[e, sl] = rows_v[e, sl] * s
                        return c2

                    lax.fori_loop(0, EB // 16, scale_16, 0)
                    pltpu.sync_copy(rows_v, accum.at[dst_v], add=True)
                    return carry2

                lax.fori_loop(0, NB, edge_block, 0)

            plsc.subcore_barrier()

            @pl.when(active)
            def _():
                def drain_piece(p, c3):
                    pr = row0 + p * EB
                    pltpu.sync_copy(accum.at[pl.ds(pr, EB)], rows_v)

                    def elu_row(rr, c2):
                        for q in range(128 // 16):
                            sl = pl.ds(q * 16, 16)
                            v = rows_v[rr, sl]
                            rows_v[rr, sl] = jnp.where(v > 0.0, v,
                                                       jnp.exp(v) - 1.0)
                        return c2

                    lax.fori_loop(0, EB, elu_row, 0)
                    pltpu.sync_copy(rows_v, out.at[pl.ds(j * NP + pr, EB)])
                    return c3

                lax.fori_loop(0, ROWS // EB, drain_piece, 0)

            return carry

        lax.fori_loop(0, rounds, round_body, 0)

    return spmm


_spmm = _make_spmm()


def _prep_edges(idx, val):
    pad = E_PAD - E
    spread = (jnp.arange(pad, dtype=jnp.int32) * 7) % N
    src = jnp.concatenate([idx[1].astype(jnp.int32), spread])
    dst = jnp.concatenate([idx[0].astype(jnp.int32), spread])
    v = jnp.concatenate([val, jnp.zeros((pad,), jnp.float32)])
    return (src.reshape(16, NB, EB), dst.reshape(16, NB, EB),
            v.reshape(16, NB, EB))


def kernel(x, adj_idx, adj_val, adj_knn_idx, adj_knn_val,
           adj_diff_idx, adj_diff_val, W1, W2, W3):
    zeros = jnp.zeros((NP, 128), jnp.float32)
    cc4 = jnp.full((16,), 4, jnp.int32)
    cc2 = jnp.full((16,), 2, jnp.int32)
    cc1 = jnp.full((16,), 1, jnp.int32)
    x = jnp.pad(x, ((0, NP - N), (0, 0)))
    sup1 = _mm_x(x, W1, 4)
    outs = []
    for idx, val in ((adj_idx, adj_val), (adj_knn_idx, adj_knn_val),
                     (adj_diff_idx, adj_diff_val)):
        s3, d3, v3 = _prep_edges(idx, val)
        h1 = _spmm(sup1, s3, d3, v3, zeros, cc4)
        sup2 = _mm_flat(h1, W2, 4, 2)
        h2 = _spmm(sup2, s3, d3, v3, zeros, cc2)
        sup3 = _mm_flat(h2, W3, 2, 1)
        h3 = _spmm(sup3, s3, d3, v3, zeros, cc1)
        outs.append(h3[:N])
    return tuple(outs)


# speedup vs baseline: 4.1815x; 2.3653x over previous
import functools

import jax
import jax.numpy as jnp
from jax import lax
from jax.experimental import pallas as pl
from jax.experimental.pallas import tpu as pltpu
from jax.experimental.pallas import tpu_sc as plsc

N = 10000
NP = 10240
E = 160000
NB = 80
EB = 128
E_PAD = 16 * NB * EB
BM = 2048
ROWS = NP // 16


def _mm_kernel(a_ref, w_ref, o_ref):
    k = pl.program_id(2)

    @pl.when(k == 0)
    def _():
        o_ref[...] = jnp.zeros_like(o_ref)

    o_ref[...] += jnp.dot(a_ref[...], w_ref[...],
                          preferred_element_type=jnp.float32)


def _mm_elu_kernel(a_ref, w_ref, o_ref):
    k = pl.program_id(2)

    @pl.when(k == 0)
    def _():
        o_ref[...] = jnp.zeros_like(o_ref)

    a = a_ref[...]
    a = jnp.where(a > 0.0, a, jnp.exp(a) - 1.0)
    o_ref[...] += jnp.dot(a, w_ref[...], preferred_element_type=jnp.float32)


def _mm_x(x, w, c_out):
    k_dim = x.shape[1]
    return pl.pallas_call(
        _mm_kernel,
        grid=(NP // BM, c_out, 1),
        in_specs=[
            pl.BlockSpec((BM, k_dim), lambda i, j, k: (i, 0)),
            pl.BlockSpec((k_dim, 128), lambda i, j, k: (0, j)),
        ],
        out_specs=pl.BlockSpec((BM, 128),
                               lambda i, j, k: (j * (NP // BM) + i, 0)),
        out_shape=jax.ShapeDtypeStruct((4 * NP, 128), jnp.float32),
    )(x, w)


def _mm_flat(h, w, c_in, c_out):
    return pl.pallas_call(
        _mm_elu_kernel,
        grid=(NP // BM, c_out, c_in),
        in_specs=[
            pl.BlockSpec((BM, 128), lambda i, j, k: (k * (NP // BM) + i, 0)),
            pl.BlockSpec((128, 128), lambda i, j, k: (k, j)),
        ],
        out_specs=pl.BlockSpec((BM, 128),
                               lambda i, j, k: (j * (NP // BM) + i, 0)),
        out_shape=jax.ShapeDtypeStruct((4 * NP, 128), jnp.float32),
    )(h, w)


def _elu_kernel(a_ref, o_ref):
    a = a_ref[...]
    o_ref[...] = jnp.where(a > 0.0, a, jnp.exp(a) - 1.0)


def _elu(h):
    return pl.pallas_call(
        _elu_kernel,
        grid=(NP // BM,),
        in_specs=[pl.BlockSpec((BM, 128), lambda i: (i, 0))],
        out_specs=pl.BlockSpec((BM, 128), lambda i: (i, 0)),
        out_shape=jax.ShapeDtypeStruct((NP, 128), jnp.float32),
    )(h)


def _make_spmm():
    mesh = plsc.VectorSubcoreMesh(core_axis_name="c", subcore_axis_name="s")

    @functools.partial(
        pl.kernel,
        mesh=mesh,
        out_type=jax.ShapeDtypeStruct((4 * NP, 128), jnp.float32),
        scratch_types=[
            [pltpu.VMEM((2 * EB,), jnp.float32)] * 4,
            [pltpu.VMEM((EB,), jnp.int32)] * 2,
            [pltpu.VMEM((EB,), jnp.int32)] * 4,
            [pltpu.VMEM((EB, 128), jnp.float32)] * 2,
            pltpu.VMEM((16,), jnp.int32),
            pltpu.VMEM_SHARED((NP, 128), jnp.float32),
            [pltpu.SemaphoreType.DMA] * 4,
            [pltpu.SemaphoreType.DMA] * 4,
            [pltpu.SemaphoreType.DMA] * 2,
            [pltpu.SemaphoreType.DMA] * 2,
        ],
    )
    def spmm(sv3, dst3, sup, zeros_hbm, cc_hbm, out,
             meta, idxb, dstv, rows, cc_v, accum, msem, dsem, gsem, scsem):
        cid = lax.axis_index("c")
        sid = lax.axis_index("s")
        row0 = sid * ROWS

        pltpu.sync_copy(cc_hbm, cc_v)
        c_chunks = cc_v[pl.ds(0, 16)][0]
        rounds = (c_chunks + 1) // 2

        def adjust(ms, islot, off):
            for q in range(EB // 16):
                sl = pl.ds(q * 16, 16)
                idxb[islot][sl] = meta[ms][sl].astype(jnp.int32) + off

        def scale(rs, ms):
            def scale16(e16, c2):
                iv = meta[ms][pl.ds(EB + e16 * 16, 16)]
                fv = plsc.bitcast(iv, jnp.float32)
                for i in range(16):
                    sc = fv[i]
                    e = e16 * 16 + i
                    for q in range(128 // 16):
                        sl = pl.ds(q * 16, 16)
                        rows[rs][e, sl] = rows[rs][e, sl] * sc
                return c2

            lax.fori_loop(0, EB // 16, scale16, 0)

        def round_body(r, carry):
            j = r * 2 + cid
            active = j < c_chunks
            off = j * NP

            pltpu.sync_copy(zeros_hbm.at[pl.ds(row0, ROWS)],
                            accum.at[pl.ds(row0, ROWS)])
            plsc.subcore_barrier()

            @pl.when(active)
            def _():
                pltpu.async_copy(dst3.at[sid, 0], dstv[0], dsem[0])
                pltpu.async_copy(dst3.at[sid, 1], dstv[1], dsem[1])
                pltpu.async_copy(sv3.at[sid, 0], meta[0], msem[0])
                pltpu.async_copy(sv3.at[sid, 1], meta[1], msem[1])
                pltpu.make_async_copy(sv3.at[sid, 0], meta[0], msem[0]).wait()
                adjust(0, 0, off)
                pltpu.async_copy(sup.at[idxb[0]], rows[0], gsem[0])

                def block(bq, s, first, nog1, nog2):
                    b = bq * 4 + s
                    rs = s & 1
                    ro = rs ^ 1
                    m1 = (s + 1) & 3
                    if not nog1:
                        pltpu.make_async_copy(sv3.at[sid, b + 1], meta[m1],
                                              msem[m1]).wait()
                        adjust(m1, ro, off)
                        if not first:
                            pltpu.make_async_copy(
                                rows[ro], accum.at[dstv[(s - 1) & 3]],
                                scsem[ro]).wait()
                        pltpu.async_copy(sup.at[idxb[ro]], rows[ro],
                                         gsem[ro])
                    if not nog2:
                        pltpu.async_copy(dst3.at[sid, b + 2],
                                         dstv[(s + 2) & 3], dsem[(s + 2) & 3])
                        pltpu.async_copy(sv3.at[sid, b + 2],
                                         meta[(s + 2) & 3], msem[(s + 2) & 3])
                    pltpu.make_async_copy(sup.at[idxb[rs]], rows[rs],
                                          gsem[rs]).wait()
                    scale(rs, s)
                    pltpu.make_async_copy(dst3.at[sid, b], dstv[s],
                                          dsem[s]).wait()
                    pltpu.async_copy(rows[rs], accum.at[dstv[s]],
                                     scsem[rs], add=True)

                for s in range(4):
                    block(0, s, first=(s == 0), nog1=False, nog2=False)

                def quad(bq, c2):
                    for s in range(4):
                        block(bq, s, first=False, nog1=False, nog2=False)
                    return c2

                lax.fori_loop(1, NB // 4 - 1, quad, 0)

                for s in range(4):
                    b = NB - 4 + s
                    block(NB // 4 - 1, s, first=False,
                          nog1=(b + 1 >= NB), nog2=(b + 2 >= NB))

                pltpu.make_async_copy(rows[0], accum.at[dstv[2]],
                                      scsem[0]).wait()
                pltpu.make_async_copy(rows[1], accum.at[dstv[3]],
                                      scsem[1]).wait()

            plsc.subcore_barrier()

            @pl.when(active)
            def _():
                pltpu.sync_copy(accum.at[pl.ds(row0, ROWS)],
                                out.at[pl.ds(j * NP + row0, ROWS)])

            return carry

        lax.fori_loop(0, rounds, round_body, 0)

    return spmm


_spmm = _make_spmm()


def _prep_edges(idx, val):
    pad = E_PAD - E
    spread = (jnp.arange(pad, dtype=jnp.int32) * 7) % N
    src = jnp.concatenate([idx[1].astype(jnp.int32), spread])
    dst = jnp.concatenate([idx[0].astype(jnp.int32), spread])
    v = jnp.concatenate([val, jnp.zeros((pad,), jnp.float32)])
    sv = jnp.stack([src.astype(jnp.float32).reshape(16, NB, EB),
                    v.reshape(16, NB, EB)],
                   axis=2).reshape(16, NB, 2 * EB)
    return sv, dst.reshape(16, NB, EB)


def kernel(x, adj_idx, adj_val, adj_knn_idx, adj_knn_val,
           adj_diff_idx, adj_diff_val, W1, W2, W3):
    zeros = jnp.zeros((NP, 128), jnp.float32)
    cc4 = jnp.full((16,), 4, jnp.int32)
    cc2 = jnp.full((16,), 2, jnp.int32)
    cc1 = jnp.full((16,), 1, jnp.int32)
    x = jnp.pad(x, ((0, NP - N), (0, 0)))
    sup1 = _mm_x(x, W1, 4)
    outs = []
    for idx, val in ((adj_idx, adj_val), (adj_knn_idx, adj_knn_val),
                     (adj_diff_idx, adj_diff_val)):
        sv3, d3 = _prep_edges(idx, val)
        h1 = _spmm(sv3, d3, sup1, zeros, cc4)
        sup2 = _mm_flat(h1, W2, 4, 2)
        h2 = _spmm(sv3, d3, sup2, zeros, cc2)
        sup3 = _mm_flat(h2, W3, 2, 1)
        h3 = _spmm(sv3, d3, sup3, zeros, cc1)
        outs.append(_elu(h3)[:N])
    return tuple(outs)


# speedup vs baseline: 4.6619x; 1.1149x over previous
import functools

import jax
import jax.numpy as jnp
from jax import lax
from jax.experimental import pallas as pl
from jax.experimental.pallas import tpu as pltpu
from jax.experimental.pallas import tpu_sc as plsc

N = 10000
NP = 10240
E = 160000
NB = 90
EB = 112
E_PAD = 16 * NB * EB
BM = 2048
ROWS = NP // 16


def _mm_kernel(a_ref, w_ref, o_ref):
    k = pl.program_id(2)

    @pl.when(k == 0)
    def _():
        o_ref[...] = jnp.zeros_like(o_ref)

    o_ref[...] += jnp.dot(a_ref[...], w_ref[...],
                          preferred_element_type=jnp.float32)


def _mm_elu_kernel(a_ref, w_ref, o_ref):
    k = pl.program_id(2)

    @pl.when(k == 0)
    def _():
        o_ref[...] = jnp.zeros_like(o_ref)

    a = a_ref[...]
    a = jnp.where(a > 0.0, a, jnp.exp(a) - 1.0)
    o_ref[...] += jnp.dot(a, w_ref[...], preferred_element_type=jnp.float32)


def _mm_x(x, w, c_out):
    k_dim = x.shape[1]
    return pl.pallas_call(
        _mm_kernel,
        grid=(NP // BM, c_out, 1),
        in_specs=[
            pl.BlockSpec((BM, k_dim), lambda i, j, k: (i, 0)),
            pl.BlockSpec((k_dim, 128), lambda i, j, k: (0, j)),
        ],
        out_specs=pl.BlockSpec((BM, 128),
                               lambda i, j, k: (j * (NP // BM) + i, 0)),
        out_shape=jax.ShapeDtypeStruct((4 * NP, 128), jnp.float32),
    )(x, w)


def _mm_flat(h, w, c_in, c_out):
    return pl.pallas_call(
        _mm_elu_kernel,
        grid=(NP // BM, c_out, c_in),
        in_specs=[
            pl.BlockSpec((BM, 128), lambda i, j, k: (k * (NP // BM) + i, 0)),
            pl.BlockSpec((128, 128), lambda i, j, k: (k, j)),
        ],
        out_specs=pl.BlockSpec((BM, 128),
                               lambda i, j, k: (j * (NP // BM) + i, 0)),
        out_shape=jax.ShapeDtypeStruct((4 * NP, 128), jnp.float32),
    )(h, w)


def _elu_kernel(a_ref, o_ref):
    a = a_ref[...]
    o_ref[...] = jnp.where(a > 0.0, a, jnp.exp(a) - 1.0)


def _elu(h):
    return pl.pallas_call(
        _elu_kernel,
        grid=(NP // BM,),
        in_specs=[pl.BlockSpec((BM, 128), lambda i: (i, 0))],
        out_specs=pl.BlockSpec((BM, 128), lambda i: (i, 0)),
        out_shape=jax.ShapeDtypeStruct((NP, 128), jnp.float32),
    )(h)


def _make_spmm():
    mesh = plsc.VectorSubcoreMesh(core_axis_name="c", subcore_axis_name="s")

    @functools.partial(
        pl.kernel,
        mesh=mesh,
        out_type=jax.ShapeDtypeStruct((4 * NP, 128), jnp.float32),
        scratch_types=[
            [pltpu.VMEM((2 * EB,), jnp.float32)] * 3,
            [pltpu.VMEM((EB,), jnp.int32)] * 3,
            [pltpu.VMEM((EB,), jnp.int32)] * 3,
            [pltpu.VMEM((EB, 128), jnp.float32)] * 3,
            pltpu.VMEM((16,), jnp.int32),
            pltpu.VMEM_SHARED((NP, 128), jnp.float32),
            [pltpu.SemaphoreType.DMA] * 3,
            [pltpu.SemaphoreType.DMA] * 3,
            [pltpu.SemaphoreType.DMA] * 3,
            [pltpu.SemaphoreType.DMA] * 3,
        ],
    )
    def spmm(sv3, dst3, sup, zeros_hbm, cc_hbm, out,
             meta, idxb, dstv, rows, cc_v, accum, msem, dsem, gsem, scsem):
        cid = lax.axis_index("c")
        sid = lax.axis_index("s")
        row0 = sid * ROWS

        pltpu.sync_copy(cc_hbm, cc_v)
        c_chunks = cc_v[pl.ds(0, 16)][0]
        rounds = (c_chunks + 1) // 2

        def adjust(ms, islot, off):
            for q in range(EB // 16):
                sl = pl.ds(q * 16, 16)
                idxb[islot][sl] = meta[ms][sl].astype(jnp.int32) + off

        def scale(rs, ms):
            def scale16(e16, c2):
                fv = meta[ms][pl.ds(EB + e16 * 16, 16)]
                for i in range(16):
                    sc = fv[i]
                    e = e16 * 16 + i
                    for q in range(128 // 16):
                        sl = pl.ds(q * 16, 16)
                        rows[rs][e, sl] = rows[rs][e, sl] * sc
                return c2

            lax.fori_loop(0, EB // 16, scale16, 0)

        def round_body(r, carry):
            j = r * 2 + cid
            active = j < c_chunks
            off = j * NP

            pltpu.sync_copy(zeros_hbm.at[pl.ds(row0, ROWS)],
                            accum.at[pl.ds(row0, ROWS)])
            plsc.subcore_barrier()

            @pl.when(active)
            def _():
                pltpu.async_copy(sv3.at[sid, 0], meta[0], msem[0])
                pltpu.async_copy(sv3.at[sid, 1], meta[1], msem[1])
                pltpu.async_copy(dst3.at[sid, 0], dstv[0], dsem[0])
                pltpu.make_async_copy(sv3.at[sid, 0], meta[0], msem[0]).wait()
                adjust(0, 0, off)
                pltpu.async_copy(sup.at[idxb[0]], rows[0], gsem[0])

                def block(bq, s, first2, nog1, nog2):
                    b = bq * 3 + s
                    s1 = (s + 1) % 3
                    s2 = (s + 2) % 3
                    if not nog1:
                        pltpu.make_async_copy(sv3.at[sid, b + 1], meta[s1],
                                              msem[s1]).wait()
                        adjust(s1, s1, off)
                        if not first2:
                            pltpu.make_async_copy(
                                rows[s1], accum.at[dstv[s1]],
                                scsem[s1]).wait()
                        pltpu.async_copy(sup.at[idxb[s1]], rows[s1],
                                         gsem[s1])
                        pltpu.async_copy(dst3.at[sid, b + 1], dstv[s1],
                                         dsem[s1])
                    if not nog2:
                        pltpu.async_copy(sv3.at[sid, b + 2], meta[s2],
                                         msem[s2])
                    pltpu.make_async_copy(sup.at[idxb[s]], rows[s],
                                          gsem[s]).wait()
                    scale(s, s)
                    pltpu.make_async_copy(dst3.at[sid, b], dstv[s],
                                          dsem[s]).wait()
                    pltpu.async_copy(rows[s], accum.at[dstv[s]],
                                     scsem[s], add=True)

                for s in range(3):
                    block(0, s, first2=(s < 2), nog1=False, nog2=False)

                def triple(bq, c2):
                    for s in range(3):
                        block(bq, s, first2=False, nog1=False, nog2=False)
                    return c2

                lax.fori_loop(1, NB // 3 - 1, triple, 0)

                for s in range(3):
                    b = NB - 3 + s
                    block(NB // 3 - 1, s, first2=False,
                          nog1=(b + 1 >= NB), nog2=(b + 2 >= NB))

                for s in range(3):
                    pltpu.make_async_copy(rows[s], accum.at[dstv[s]],
                                          scsem[s]).wait()

            plsc.subcore_barrier()

            @pl.when(active)
            def _():
                pltpu.sync_copy(accum.at[pl.ds(row0, ROWS)],
                                out.at[pl.ds(j * NP + row0, ROWS)])

            return carry

        lax.fori_loop(0, rounds, round_body, 0)

    return spmm


_spmm = _make_spmm()


def _prep_edges(idx, val):
    pad = E_PAD - E
    spread = (jnp.arange(pad, dtype=jnp.int32) * 7) % N
    src = jnp.concatenate([idx[1].astype(jnp.int32), spread])
    dst = jnp.concatenate([idx[0].astype(jnp.int32), spread])
    v = jnp.concatenate([val, jnp.zeros((pad,), jnp.float32)])
    sv = jnp.stack([src.astype(jnp.float32).reshape(16, NB, EB),
                    v.reshape(16, NB, EB)],
                   axis=2).reshape(16, NB, 2 * EB)
    return sv, dst.reshape(16, NB, EB)


def kernel(x, adj_idx, adj_val, adj_knn_idx, adj_knn_val,
           adj_diff_idx, adj_diff_val, W1, W2, W3):
    zeros = jnp.zeros((NP, 128), jnp.float32)
    cc4 = jnp.full((16,), 4, jnp.int32)
    cc2 = jnp.full((16,), 2, jnp.int32)
    cc1 = jnp.full((16,), 1, jnp.int32)
    x = jnp.pad(x, ((0, NP - N), (0, 0)))
    sup1 = _mm_x(x, W1, 4)
    outs = []
    for idx, val in ((adj_idx, adj_val), (adj_knn_idx, adj_knn_val),
                     (adj_diff_idx, adj_diff_val)):
        sv3, d3 = _prep_edges(idx, val)
        h1 = _spmm(sv3, d3, sup1, zeros, cc4)
        sup2 = _mm_flat(h1, W2, 4, 2)
        h2 = _spmm(sv3, d3, sup2, zeros, cc2)
        sup3 = _mm_flat(h2, W3, 2, 1)
        h3 = _spmm(sv3, d3, sup3, zeros, cc1)
        outs.append(_elu(h3)[:N])
    return tuple(outs)
